# baseline (speedup 1.0000x reference)
import functools

import jax
import jax.numpy as jnp
from jax import lax
from jax.experimental import pallas as pl
from jax.experimental.pallas import tpu as pltpu
from jax.experimental.pallas import tpu_sc as plsc

NUM_CORES = 2
NUM_SUBCORES = 16
NW = NUM_CORES * NUM_SUBCORES
B = 16384
D = 128
RPW = B // NW
CHUNK = 128
CPW = RPW // CHUNK
PAIR = 2 * CHUNK
PPT = CPW // 2
NPAIR = 2 * PPT
NBUF = 3

RELK = 1024
TCB = 2048


def _ht_body(h_idx, t_idx, ent_hbm, h_out, t_out, idx_v, bufs, gsem, wsem):
    wid = lax.axis_index("s") * NUM_CORES + lax.axis_index("c")
    base = wid * RPW
    row0 = wid * CPW

    for tbl_i, idx_hbm in enumerate((h_idx, t_idx)):
        pltpu.sync_copy(idx_hbm.at[pl.ds(row0, CPW)], idx_v.at[tbl_i])

    outs = (h_out, t_out)

    def start_pair(p):
        tbl_i, half = divmod(p, PPT)
        return [pltpu.async_copy(
                    ent_hbm.at[idx_v.at[tbl_i].at[half * 2 + k]],
                    bufs.at[p % NBUF].at[pl.ds(k * CHUNK, CHUNK)], gsem)
                for k in range(2)]

    def start_wb(p):
        tbl_i, half = divmod(p, PPT)
        return pltpu.async_copy(
            bufs.at[p % NBUF], outs[tbl_i].at[pl.ds(base + half * PAIR, PAIR)], wsem)

    g = [None] * NPAIR
    w = [None] * NPAIR
    for p in range(NBUF):
        g[p] = start_pair(p)
    for p in range(NPAIR):
        if 0 < p <= NPAIR - NBUF:
            w[p - 1].wait()
            g[p - 1 + NBUF] = start_pair(p - 1 + NBUF)
        for hnd in g[p]:
            hnd.wait()
        w[p] = start_wb(p)
    for p in range(max(0, NPAIR - NBUF), NPAIR):
        w[p].wait()


def _ht_gather(h2, t2, entity_emb):
    mesh = plsc.VectorSubcoreMesh(core_axis_name="c", subcore_axis_name="s")
    out_t = (jax.ShapeDtypeStruct((B, D), jnp.float32),) * 2
    run = functools.partial(
        pl.kernel, mesh=mesh,
        out_type=out_t,
        scratch_types=[
            pltpu.VMEM((2, CPW, CHUNK), jnp.int32),
            pltpu.VMEM((NBUF, PAIR, D), jnp.float32),
            pltpu.SemaphoreType.DMA,
            pltpu.SemaphoreType.DMA,
        ],
    )(_ht_body)
    return run(h2, t2, entity_emb)


def _rel_body(idx_ref, tbl_ref, out_ref):
    kcol = lax.broadcasted_iota(jnp.int16, (RELK, 1), 0)
    tbl = tbl_ref[...]
    for s in range(0, 16, 2):
        oh = jnp.concatenate(
            [(kcol == idx_ref[s + k:s + k + 1, :]).astype(jnp.bfloat16)
             for k in range(2)], axis=1)
        out_ref[pl.ds(s * 128, 256), :] = lax.dot_general(
            oh, tbl, (((0,), (0,)), ((), ())),
            preferred_element_type=jnp.float32)


def _rel_gather(r2, rel_padded):
    return pl.pallas_call(
        _rel_body,
        grid=(B // TCB,),
        in_specs=[pl.BlockSpec((16, 128), lambda i: (i, 0)),
                  pl.BlockSpec((RELK, D), lambda i: (0, 0))],
        out_specs=pl.BlockSpec((TCB, D), lambda i: (i, 0)),
        out_shape=jax.ShapeDtypeStruct((B, D), jnp.float32),
    )(r2, rel_padded)


def kernel(h, r, t, entity_emb, relation_emb):
    h2 = h.astype(jnp.int32).reshape(B // CHUNK, CHUNK)
    t2 = t.astype(jnp.int32).reshape(B // CHUNK, CHUNK)
    h_emb, t_emb = _ht_gather(h2, t2, entity_emb)
    rel_padded = jnp.pad(
        relation_emb, ((0, RELK - relation_emb.shape[0]), (0, 0))
    ).astype(jnp.bfloat16)
    r_emb = _rel_gather(r.astype(jnp.int16).reshape(B // 128, 128), rel_padded)
    return (h_emb, r_emb, t_emb)


# speedup vs baseline: 1.0027x; 1.0027x over previous
import functools

import jax
import jax.numpy as jnp
from jax import lax
from jax.experimental import pallas as pl
from jax.experimental.pallas import tpu as pltpu
from jax.experimental.pallas import tpu_sc as plsc

NUM_CORES = 2
NUM_SUBCORES = 16
NW = NUM_CORES * NUM_SUBCORES
B = 16384
D = 128
RPW = B // NW
CHUNK = 128
CPW = RPW // CHUNK
PAIR = 2 * CHUNK
PPT = CPW // 2
NPAIR = 2 * PPT
NBUF = 3

RELK = 1024
TCB = 2048


def _ht_body(h_idx, t_idx, ent_hbm, h_out, t_out, idx_v, bufs, gsem, wsem):
    wid = lax.axis_index("s") * NUM_CORES + lax.axis_index("c")
    base = wid * RPW
    row0 = wid * CPW

    for tbl_i, idx_hbm in enumerate((h_idx, t_idx)):
        pltpu.sync_copy(idx_hbm.at[pl.ds(row0, CPW)], idx_v.at[tbl_i])

    outs = (h_out, t_out)

    def start_pair(p):
        tbl_i, half = divmod(p, PPT)
        return [pltpu.async_copy(
                    ent_hbm.at[idx_v.at[tbl_i].at[half * 2 + k]],
                    bufs.at[p % NBUF].at[pl.ds(k * CHUNK, CHUNK)], gsem)
                for k in range(2)]

    def start_wb(p):
        tbl_i, half = divmod(p, PPT)
        return pltpu.async_copy(
            bufs.at[p % NBUF], outs[tbl_i].at[pl.ds(base + half * PAIR, PAIR)], wsem)

    g = [None] * NPAIR
    w = [None] * NPAIR
    for p in range(NBUF):
        g[p] = start_pair(p)
    for p in range(NPAIR):
        if 0 < p <= NPAIR - NBUF:
            w[p - 1].wait()
            g[p - 1 + NBUF] = start_pair(p - 1 + NBUF)
        for hnd in g[p]:
            hnd.wait()
        w[p] = start_wb(p)
    for p in range(max(0, NPAIR - NBUF), NPAIR):
        w[p].wait()


def _ht_gather(h2, t2, entity_emb):
    mesh = plsc.VectorSubcoreMesh(core_axis_name="c", subcore_axis_name="s")
    out_t = (jax.ShapeDtypeStruct((B, D), jnp.float32),) * 2
    run = functools.partial(
        pl.kernel, mesh=mesh,
        out_type=out_t,
        scratch_types=[
            pltpu.VMEM((2, CPW, CHUNK), jnp.int32),
            pltpu.VMEM((NBUF, PAIR, D), jnp.float32),
            pltpu.SemaphoreType.DMA,
            pltpu.SemaphoreType.DMA,
        ],
    )(_ht_body)
    return run(h2, t2, entity_emb)


def _rel_body(idx_ref, tbl_ref, out_ref):
    kcol = lax.broadcasted_iota(jnp.int16, (RELK, 1), 0)
    tbl = tbl_ref[...]
    for s in range(16):
        row = idx_ref[s:s + 1, :]
        oh = (kcol == row).astype(jnp.bfloat16)
        out_ref[pl.ds(s * 128, 128), :] = lax.dot_general(
            oh, tbl, (((0,), (0,)), ((), ())),
            preferred_element_type=jnp.float32)


def _rel_gather(r2, rel_padded):
    return pl.pallas_call(
        _rel_body,
        grid=(B // TCB,),
        in_specs=[pl.BlockSpec((16, 128), lambda i: (i, 0)),
                  pl.BlockSpec((RELK, D), lambda i: (0, 0))],
        out_specs=pl.BlockSpec((TCB, D), lambda i: (i, 0)),
        out_shape=jax.ShapeDtypeStruct((B, D), jnp.float32),
    )(r2, rel_padded)


def kernel(h, r, t, entity_emb, relation_emb):
    h2 = h.astype(jnp.int32).reshape(B // CHUNK, CHUNK)
    t2 = t.astype(jnp.int32).reshape(B // CHUNK, CHUNK)
    h_emb, t_emb = _ht_gather(h2, t2, entity_emb)
    rel_padded = jnp.pad(
        relation_emb, ((0, RELK - relation_emb.shape[0]), (0, 0))
    ).astype(jnp.bfloat16)
    r_emb = _rel_gather(r.astype(jnp.int16).reshape(B // 128, 128), rel_padded)
    return (h_emb, r_emb, t_emb)


# speedup vs baseline: 1.0810x; 1.0781x over previous
import functools

import jax
import jax.numpy as jnp
from jax import lax
from jax.experimental import pallas as pl
from jax.experimental.pallas import tpu as pltpu
from jax.experimental.pallas import tpu_sc as plsc

NUM_CORES = 2
NUM_SUBCORES = 16
NW = NUM_CORES * NUM_SUBCORES
B = 16384
D = 128
RPW = B // NW
CHUNK = 128
CPW = RPW // CHUNK
NCH = 2 * CPW
NBUF = 7

RELK = 1024
TCB = 1024


def _ht_body(h_idx, t_idx, ent_hbm, h_out, t_out, idx_v, bufs, gsem, wsem):
    wid = lax.axis_index("s") * NUM_CORES + lax.axis_index("c")
    base = wid * RPW
    row0 = wid * CPW

    for tbl_i, idx_hbm in enumerate((h_idx, t_idx)):
        pltpu.sync_copy(idx_hbm.at[pl.ds(row0, CPW)], idx_v.at[tbl_i])

    outs = (h_out, t_out)

    def start_gather(c):
        tbl_i, j = divmod(c, CPW)
        return pltpu.async_copy(
            ent_hbm.at[idx_v.at[tbl_i].at[j]], bufs.at[c % NBUF], gsem)

    def start_wb(c):
        tbl_i, j = divmod(c, CPW)
        return pltpu.async_copy(
            bufs.at[c % NBUF], outs[tbl_i].at[pl.ds(base + j * CHUNK, CHUNK)], wsem)

    g = [None] * NCH
    w = [None] * NCH
    for c in range(NBUF):
        g[c] = start_gather(c)
    for c in range(NCH):
        if 0 < c <= NCH - NBUF:
            w[c - 1].wait()
            g[c - 1 + NBUF] = start_gather(c - 1 + NBUF)
        g[c].wait()
        w[c] = start_wb(c)
    for c in range(max(0, NCH - NBUF), NCH):
        w[c].wait()


def _ht_gather(h2, t2, entity_emb):
    mesh = plsc.VectorSubcoreMesh(core_axis_name="c", subcore_axis_name="s")
    out_t = (jax.ShapeDtypeStruct((B, D), jnp.float32),) * 2
    run = functools.partial(
        pl.kernel, mesh=mesh,
        out_type=out_t,
        scratch_types=[
            pltpu.VMEM((2, CPW, CHUNK), jnp.int32),
            pltpu.VMEM((NBUF, CHUNK, D), jnp.float32),
            pltpu.SemaphoreType.DMA,
            pltpu.SemaphoreType.DMA,
        ],
    )(_ht_body)
    return run(h2, t2, entity_emb)


def _rel_body(idx_ref, tbl_ref, out_ref):
    kcol = lax.broadcasted_iota(jnp.int32, (RELK, 1), 0)
    tbl = tbl_ref[...]
    for s in range(8):
        row = idx_ref[s:s + 1, :]
        oh = (kcol == row).astype(jnp.bfloat16)
        out_ref[pl.ds(s * 128, 128), :] = lax.dot_general(
            oh, tbl, (((0,), (0,)), ((), ())),
            preferred_element_type=jnp.float32)


def _rel_gather(r2, rel_padded):
    return pl.pallas_call(
        _rel_body,
        grid=(B // TCB,),
        in_specs=[pl.BlockSpec((8, 128), lambda i: (i, 0)),
                  pl.BlockSpec((RELK, D), lambda i: (0, 0))],
        out_specs=pl.BlockSpec((TCB, D), lambda i: (i, 0)),
        out_shape=jax.ShapeDtypeStruct((B, D), jnp.float32),
    )(r2, rel_padded)


def kernel(h, r, t, entity_emb, relation_emb):
    h2 = h.astype(jnp.int32).reshape(B // CHUNK, CHUNK)
    t2 = t.astype(jnp.int32).reshape(B // CHUNK, CHUNK)
    h_emb, t_emb = _ht_gather(h2, t2, entity_emb)
    rel_padded = jnp.pad(
        relation_emb, ((0, RELK - relation_emb.shape[0]), (0, 0))
    ).astype(jnp.bfloat16)
    r_emb = _rel_gather(r.astype(jnp.int32).reshape(B // 128, 128), rel_padded)
    return (h_emb, r_emb, t_emb)


# speedup vs baseline: 1.0908x; 1.0090x over previous
import functools

import jax
import jax.numpy as jnp
from jax import lax
from jax.experimental import pallas as pl
from jax.experimental.pallas import tpu as pltpu
from jax.experimental.pallas import tpu_sc as plsc

NUM_CORES = 2
NUM_SUBCORES = 16
NW = NUM_CORES * NUM_SUBCORES
B = 16384
D = 128
RPW = B // NW
CHUNK = 128
CPW = RPW // CHUNK
PAIR = 2 * CHUNK
PPT = CPW // 2
NPAIR = 2 * PPT
NBUF = 3

RELK = 1024
TCB = 1024


def _ht_body(h_idx, t_idx, ent_hbm, h_out, t_out, idx_v, bufs, gsem, wsem):
    wid = lax.axis_index("s") * NUM_CORES + lax.axis_index("c")
    base = wid * RPW
    row0 = wid * CPW

    for tbl_i, idx_hbm in enumerate((h_idx, t_idx)):
        pltpu.sync_copy(idx_hbm.at[pl.ds(row0, CPW)], idx_v.at[tbl_i])

    outs = (h_out, t_out)

    def start_pair(p):
        tbl_i, half = divmod(p, PPT)
        return [pltpu.async_copy(
                    ent_hbm.at[idx_v.at[tbl_i].at[half * 2 + k]],
                    bufs.at[p % NBUF].at[pl.ds(k * CHUNK, CHUNK)], gsem)
                for k in range(2)]

    def start_wb(p):
        tbl_i, half = divmod(p, PPT)
        return pltpu.async_copy(
            bufs.at[p % NBUF], outs[tbl_i].at[pl.ds(base + half * PAIR, PAIR)], wsem)

    g = [None] * NPAIR
    w = [None] * NPAIR
    for p in range(NBUF):
        g[p] = start_pair(p)
    for p in range(NPAIR):
        if 0 < p <= NPAIR - NBUF:
            w[p - 1].wait()
            g[p - 1 + NBUF] = start_pair(p - 1 + NBUF)
        for hnd in g[p]:
            hnd.wait()
        w[p] = start_wb(p)
    for p in range(max(0, NPAIR - NBUF), NPAIR):
        w[p].wait()


def _ht_gather(h2, t2, entity_emb):
    mesh = plsc.VectorSubcoreMesh(core_axis_name="c", subcore_axis_name="s")
    out_t = (jax.ShapeDtypeStruct((B, D), jnp.float32),) * 2
    run = functools.partial(
        pl.kernel, mesh=mesh,
        out_type=out_t,
        scratch_types=[
            pltpu.VMEM((2, CPW, CHUNK), jnp.int32),
            pltpu.VMEM((NBUF, PAIR, D), jnp.float32),
            pltpu.SemaphoreType.DMA,
            pltpu.SemaphoreType.DMA,
        ],
    )(_ht_body)
    return run(h2, t2, entity_emb)


def _rel_body(idx_ref, tbl_ref, out_ref):
    kcol = lax.broadcasted_iota(jnp.int32, (RELK, 1), 0)
    tbl = tbl_ref[...]
    for s in range(0, 8, 2):
        row2 = idx_ref[s:s + 2, :].reshape(1, 256)
        oh = (kcol == row2).astype(jnp.bfloat16)
        out_ref[pl.ds(s * 128, 256), :] = lax.dot_general(
            oh, tbl, (((0,), (0,)), ((), ())),
            preferred_element_type=jnp.float32)


def _rel_gather(r2, rel_padded):
    return pl.pallas_call(
        _rel_body,
        grid=(B // TCB,),
        in_specs=[pl.BlockSpec((8, 128), lambda i: (i, 0)),
                  pl.BlockSpec((RELK, D), lambda i: (0, 0))],
        out_specs=pl.BlockSpec((TCB, D), lambda i: (i, 0)),
        out_shape=jax.ShapeDtypeStruct((B, D), jnp.float32),
    )(r2, rel_padded)


def kernel(h, r, t, entity_emb, relation_emb):
    h2 = h.astype(jnp.int32).reshape(B // CHUNK, CHUNK)
    t2 = t.astype(jnp.int32).reshape(B // CHUNK, CHUNK)
    h_emb, t_emb = _ht_gather(h2, t2, entity_emb)
    rel_padded = jnp.pad(
        relation_emb, ((0, RELK - relation_emb.shape[0]), (0, 0))
    ).astype(jnp.bfloat16)
    r_emb = _rel_gather(r.astype(jnp.int32).reshape(B // 128, 128), rel_padded)
    return (h_emb, r_emb, t_emb)


# speedup vs baseline: 1.1611x; 1.0645x over previous
import functools

import jax
import jax.numpy as jnp
from jax import lax
from jax.experimental import pallas as pl
from jax.experimental.pallas import tpu as pltpu
from jax.experimental.pallas import tpu_sc as plsc

NUM_CORES = 2
NUM_SUBCORES = 16
NW = NUM_CORES * NUM_SUBCORES
B = 16384
D = 128
RPW = B // NW
CHUNK = 128
CPW = RPW // CHUNK
PAIR = 2 * CHUNK
PPT = CPW // 2
NPAIR = 2 * PPT
NBUF = 3

RELK = 1024
TCB = 2048


def _ht_body(h_idx, t_idx, ent_hbm, h_out, t_out, idx_v, bufs, gsem, wsem):
    wid = lax.axis_index("s") * NUM_CORES + lax.axis_index("c")
    base = wid * RPW
    row0 = wid * CPW

    for tbl_i, idx_hbm in enumerate((h_idx, t_idx)):
        pltpu.sync_copy(idx_hbm.at[pl.ds(row0, CPW)], idx_v.at[tbl_i])

    outs = (h_out, t_out)

    def start_pair(p):
        tbl_i, half = divmod(p, PPT)
        return [pltpu.async_copy(
                    ent_hbm.at[idx_v.at[tbl_i].at[half * 2 + k]],
                    bufs.at[p % NBUF].at[pl.ds(k * CHUNK, CHUNK)], gsem)
                for k in range(2)]

    def start_wb(p):
        tbl_i, half = divmod(p, PPT)
        return pltpu.async_copy(
            bufs.at[p % NBUF], outs[tbl_i].at[pl.ds(base + half * PAIR, PAIR)], wsem)

    g = [None] * NPAIR
    w = [None] * NPAIR
    for p in range(NBUF):
        g[p] = start_pair(p)
    for p in range(NPAIR):
        if 0 < p <= NPAIR - NBUF:
            w[p - 1].wait()
            g[p - 1 + NBUF] = start_pair(p - 1 + NBUF)
        for hnd in g[p]:
            hnd.wait()
        w[p] = start_wb(p)
    for p in range(max(0, NPAIR - NBUF), NPAIR):
        w[p].wait()


def _ht_gather(h2, t2, entity_emb):
    mesh = plsc.VectorSubcoreMesh(core_axis_name="c", subcore_axis_name="s")
    out_t = (jax.ShapeDtypeStruct((B, D), jnp.float32),) * 2
    run = functools.partial(
        pl.kernel, mesh=mesh,
        out_type=out_t,
        scratch_types=[
            pltpu.VMEM((2, CPW, CHUNK), jnp.int32),
            pltpu.VMEM((NBUF, PAIR, D), jnp.float32),
            pltpu.SemaphoreType.DMA,
            pltpu.SemaphoreType.DMA,
        ],
    )(_ht_body)
    return run(h2, t2, entity_emb)


def _rel_body(idx_ref, tbl_ref, out_ref):
    kcol = lax.broadcasted_iota(jnp.int32, (RELK, 1), 0)
    tbl = tbl_ref[...]
    for s in range(0, 16, 2):
        row2 = idx_ref[s:s + 2, :].reshape(1, 256)
        oh = (kcol == row2).astype(jnp.bfloat16)
        out_ref[pl.ds(s * 128, 256), :] = lax.dot_general(
            oh, tbl, (((0,), (0,)), ((), ())),
            preferred_element_type=jnp.float32)


def _rel_gather(r2, rel_padded):
    return pl.pallas_call(
        _rel_body,
        grid=(B // TCB,),
        in_specs=[pl.BlockSpec((16, 128), lambda i: (i, 0)),
                  pl.BlockSpec((RELK, D), lambda i: (0, 0))],
        out_specs=pl.BlockSpec((TCB, D), lambda i: (i, 0)),
        out_shape=jax.ShapeDtypeStruct((B, D), jnp.float32),
    )(r2, rel_padded)


def kernel(h, r, t, entity_emb, relation_emb):
    h2 = h.astype(jnp.int32).reshape(B // CHUNK, CHUNK)
    t2 = t.astype(jnp.int32).reshape(B // CHUNK, CHUNK)
    h_emb, t_emb = _ht_gather(h2, t2, entity_emb)
    rel_padded = jnp.pad(
        relation_emb, ((0, RELK - relation_emb.shape[0]), (0, 0))
    ).astype(jnp.bfloat16)
    r_emb = _rel_gather(r.astype(jnp.int32).reshape(B // 128, 128), rel_padded)
    return (h_emb, r_emb, t_emb)
